# baseline (speedup 1.0000x reference)
import functools

import jax
import jax.numpy as jnp
from jax import lax
from jax.experimental import pallas as pl
from jax.experimental.pallas import tpu as pltpu
from jax.experimental.pallas import tpu_sc as plsc

BATCH = 4096
DIM = 128

_NC = 2
_NS = 16
_NW = _NC * _NS
_B_PER_W = BATCH // _NW

_mesh = plsc.VectorSubcoreMesh(core_axis_name="c", subcore_axis_name="s")


_NCHUNK = 4
_CHUNK = _B_PER_W // _NCHUNK


@functools.partial(
    pl.kernel,
    mesh=_mesh,
    out_type=jax.ShapeDtypeStruct((BATCH, DIM), jnp.float32),
    scratch_types=(
        [pltpu.VMEM((_CHUNK,), jnp.int32)] * _NCHUNK
        + [pltpu.VMEM((_CHUNK, DIM), jnp.float32)] * _NCHUNK
        + [pltpu.SemaphoreType.DMA] * (2 * _NCHUNK)
    ),
)
def _sc_gather(idx_hbm, table_hbm, out_hbm, *scratch):
    idx_v = scratch[:_NCHUNK]
    rows_v = scratch[_NCHUNK:2 * _NCHUNK]
    sg = scratch[2 * _NCHUNK:3 * _NCHUNK]
    ss = scratch[3 * _NCHUNK:]
    wid = lax.axis_index("s") * _NC + lax.axis_index("c")
    base = wid * _B_PER_W
    gathers = []
    for k in range(_NCHUNK):
        pltpu.sync_copy(idx_hbm.at[pl.ds(base + k * _CHUNK, _CHUNK)], idx_v[k])
        gathers.append(pltpu.async_copy(table_hbm.at[idx_v[k]], rows_v[k], sg[k]))
    stores = []
    for k in range(_NCHUNK):
        gathers[k].wait()
        stores.append(pltpu.async_copy(
            rows_v[k], out_hbm.at[pl.ds(base + k * _CHUNK, _CHUNK)], ss[k]))
    for s in stores:
        s.wait()


def kernel(organism_index, embed_weight):
    idx = organism_index.astype(jnp.int32)
    return _sc_gather(idx, embed_weight)


# speedup vs baseline: 1.0408x; 1.0408x over previous
import functools

import jax
import jax.numpy as jnp
from jax import lax
from jax.experimental import pallas as pl
from jax.experimental.pallas import tpu as pltpu
from jax.experimental.pallas import tpu_sc as plsc

BATCH = 4096
DIM = 128

_NC = 2
_NS = 16
_NW = _NC * _NS
_B_PER_W = BATCH // _NW

_mesh = plsc.VectorSubcoreMesh(core_axis_name="c", subcore_axis_name="s")


_HALF = _B_PER_W // 2


@functools.partial(
    pl.kernel,
    mesh=_mesh,
    out_type=jax.ShapeDtypeStruct((BATCH, DIM), jnp.float32),
    scratch_types=[
        pltpu.VMEM((_B_PER_W,), jnp.int32),
        pltpu.VMEM((_HALF, DIM), jnp.float32),
        pltpu.VMEM((_HALF, DIM), jnp.float32),
        pltpu.SemaphoreType.DMA,
        pltpu.SemaphoreType.DMA,
        pltpu.SemaphoreType.DMA,
        pltpu.SemaphoreType.DMA,
    ],
)
def _sc_gather(idx_hbm, table_hbm, out_hbm,
               idx_v, rows0, rows1, sg0, sg1, ss0, ss1):
    wid = lax.axis_index("s") * _NC + lax.axis_index("c")
    base = wid * _B_PER_W
    pltpu.sync_copy(idx_hbm.at[pl.ds(base, _B_PER_W)], idx_v)
    g0 = pltpu.async_copy(table_hbm.at[idx_v.at[pl.ds(0, _HALF)]], rows0, sg0)
    g1 = pltpu.async_copy(table_hbm.at[idx_v.at[pl.ds(_HALF, _HALF)]], rows1, sg1)
    g0.wait()
    s0 = pltpu.async_copy(rows0, out_hbm.at[pl.ds(base, _HALF)], ss0)
    g1.wait()
    s1 = pltpu.async_copy(rows1, out_hbm.at[pl.ds(base + _HALF, _HALF)], ss1)
    s0.wait()
    s1.wait()


def kernel(organism_index, embed_weight):
    idx = organism_index.astype(jnp.int32)
    return _sc_gather(idx, embed_weight)
